# baseline (speedup 1.0000x reference)
import jax
import jax.numpy as jnp
from jax import lax
from jax.experimental import pallas as pl
from jax.experimental.pallas import tpu as pltpu
from jax.experimental.pallas import tpu_sc as plsc

_BB = 4
_B = 64
_P = 1024
_D = 768
_SCB = 16
_NC = 2
_NS = 16
_NW = _NC * _NS
_ROWS = _P // _NW
_CHUNKS = _D // 16


def _add_kernel(x_ref, pos_ref, o_ref):
    o_ref[...] = x_ref[...] + pos_ref[...][None, :, :]


def _tc_part(input_patch, pos_table):
    B, P, D = input_patch.shape
    return pl.pallas_call(
        _add_kernel,
        grid=(B // _BB,),
        in_specs=[
            pl.BlockSpec((_BB, P, D), lambda i: (i, 0, 0)),
            pl.BlockSpec((P, D), lambda i: (0, 0)),
        ],
        out_specs=pl.BlockSpec((_BB, P, D), lambda i: (i, 0, 0)),
        out_shape=jax.ShapeDtypeStruct((B, P, D), input_patch.dtype),
    )(input_patch, pos_table)


def _sc_add(in_hbm, pos_hbm, out_hbm, pos_buf, io_buf):
    wid = lax.axis_index("s") * _NC + lax.axis_index("c")
    row0 = wid * _ROWS
    pltpu.sync_copy(pos_hbm.at[pl.ds(row0, _ROWS), :], pos_buf)

    def batch_body(b, carry):
        pltpu.sync_copy(in_hbm.at[b, pl.ds(row0, _ROWS), :], io_buf)

        def row_body(r, c2):
            for j in range(_CHUNKS):
                sl = pl.ds(j * 16, 16)
                io_buf[r, sl] = io_buf[r, sl] + pos_buf[r, sl]
            return c2

        lax.fori_loop(0, _ROWS, row_body, 0)
        pltpu.sync_copy(io_buf, out_hbm.at[b, pl.ds(row0, _ROWS), :])
        return carry

    lax.fori_loop(0, _SCB, batch_body, 0)


def _sc_part(input_patch, pos_table):
    mesh = plsc.VectorSubcoreMesh(core_axis_name="c", subcore_axis_name="s")
    k = pl.kernel(
        _sc_add,
        mesh=mesh,
        out_type=jax.ShapeDtypeStruct((_SCB, _P, _D), jnp.float32),
        scratch_types=[
            pltpu.VMEM((_ROWS, _D), jnp.float32),
            pltpu.VMEM((_ROWS, _D), jnp.float32),
        ],
    )
    return k(input_patch, pos_table)


def kernel(input_patch, pos_table):
    tc_out = _tc_part(input_patch, pos_table)
    sc_out = _sc_part(input_patch, pos_table)
    return tc_out.at[0, 0, 0].add(0.0 * sc_out[0, 0, 0])


# speedup vs baseline: 1.4428x; 1.4428x over previous
import jax
import jax.numpy as jnp
from jax.experimental import pallas as pl
from jax.experimental.pallas import tpu as pltpu

_BB = 4


def _add_kernel(x_ref, pos_ref, o_ref):
    o_ref[...] = x_ref[...] + pos_ref[...][None, :, :]


def kernel(input_patch, pos_table):
    B, P, D = input_patch.shape
    grid = (B // _BB,)
    return pl.pallas_call(
        _add_kernel,
        grid=grid,
        in_specs=[
            pl.BlockSpec((_BB, P, D), lambda i: (i, 0, 0)),
            pl.BlockSpec((P, D), lambda i: (0, 0)),
        ],
        out_specs=pl.BlockSpec((_BB, P, D), lambda i: (i, 0, 0)),
        out_shape=jax.ShapeDtypeStruct((B, P, D), input_patch.dtype),
        compiler_params=pltpu.CompilerParams(dimension_semantics=("parallel",)),
    )(input_patch, pos_table)
